# baseline (speedup 1.0000x reference)
import functools

import jax
import jax.numpy as jnp
from jax import lax
from jax.experimental import pallas as pl
from jax.experimental.pallas import tpu as pltpu
from jax.experimental.pallas import tpu_sc as plsc

_N = 10000
_E = 320000
_NC = 2
_NS = 16
_NW = _NC * _NS
_NP = 10240
_RPT = _NP // _NS


def _make_edge_pass(D, multi_head, _B):
  _NBATCH = _E // _B
  mesh = plsc.VectorSubcoreMesh(core_axis_name="c", subcore_axis_name="s")
  denw = _NP * (8 if multi_head else 1)

  @functools.partial(
      pl.kernel,
      out_type=(
          jax.ShapeDtypeStruct((_NC, _NP, D), jnp.float32),
          jax.ShapeDtypeStruct((_NC, _NS, denw), jnp.float32),
      ),
      mesh=mesh,
      compiler_params=pltpu.CompilerParams(
          use_tc_tiling_on_sc=False, needs_layout_passes=False),
      scratch_types=(
          pltpu.VMEM_SHARED((_NP, D), jnp.float32),
          pltpu.VMEM((denw,), jnp.float32),
          pltpu.VMEM((_B,), jnp.int32),
          pltpu.VMEM((_B,), jnp.int32),
          pltpu.VMEM((_B, D), jnp.float32),
          pltpu.VMEM((_B, 16), jnp.float32),
          pltpu.VMEM((_B, 16), jnp.float32),
          pltpu.VMEM((_B, D), jnp.float32),
          pltpu.SemaphoreType.DMA,
          pltpu.SemaphoreType.DMA,
          pltpu.SemaphoreType.DMA,
      ),
  )
  def kern(th, tssrc, tsdst, src, dst, zacc, zden, accp, denp,
           acc, den, idx_src, idx_dst, hrows, ssrc, sdst, orows,
           sem_h, sem_s, sem_d):
    cid = lax.axis_index("c")
    sid = lax.axis_index("s")
    wid = sid * _NC + cid
    iota = lax.iota(jnp.int32, 16)
    pat8 = iota >> 3
    lane8 = iota & 7

    r0 = sid * _RPT
    pltpu.sync_copy(zacc.at[pl.ds(r0, _RPT)], acc.at[pl.ds(r0, _RPT)])
    pltpu.sync_copy(zden, den)
    plsc.subcore_barrier()

    nb = (_NBATCH - wid + _NW - 1) // _NW

    def batch_body(i, carry):
      b = wid + i * _NW
      e0 = b * _B
      pltpu.sync_copy(src.at[pl.ds(e0, _B)], idx_src)
      pltpu.sync_copy(dst.at[pl.ds(e0, _B)], idx_dst)
      cp_h = pltpu.async_copy(th.at[idx_src], hrows, sem_h)
      cp_s = pltpu.async_copy(tssrc.at[idx_src], ssrc, sem_s)
      cp_d = pltpu.async_copy(tsdst.at[idx_dst], sdst, sem_d)
      cp_s.wait()
      cp_d.wait()
      cp_h.wait()

      def edge_body(e, c2):
        xv = ssrc[e] + sdst[e]
        wv = jnp.exp(jnp.maximum(xv, 0.2 * xv))
        dchunk = idx_dst[pl.ds((e >> 4) * 16, 16)]
        dsp = dchunk.at[jnp.full((16,), e & 15, jnp.int32)].get(
            mode="promise_in_bounds")
        if multi_head:
          plsc.addupdate_scatter(den, [dsp * 8 + lane8], wv, mask=pat8 == 0)
        else:
          plsc.addupdate_scatter(den, [dsp], wv, mask=iota == 0)
        for c in range(D // 16):
          if multi_head:
            col = 2 * c + pat8
          else:
            col = jnp.zeros((16,), jnp.int32)
          wb = wv.at[col].get(mode="promise_in_bounds")
          orows[e, pl.ds(c * 16, 16)] = wb * hrows[e, pl.ds(c * 16, 16)]
        return c2

      lax.fori_loop(0, _B, edge_body, 0)

      pltpu.sync_copy(orows, acc.at[idx_dst], add=True)
      return carry

    lax.fori_loop(0, nb, batch_body, 0)
    plsc.subcore_barrier()

    pltpu.sync_copy(acc.at[pl.ds(r0, _RPT)], accp.at[cid, pl.ds(r0, _RPT)])
    pltpu.sync_copy(den, denp.at[cid, sid])

  return kern


_edge1 = _make_edge_pass(64, True, 32)
_edge2 = _make_edge_pass(16, False, 128)

_R = 1000
_RP = 1024


def _tc_a(x, w1cat, asrc, adst):
  def body(x_ref, w_ref, as_ref, ad_ref, h_ref, ss_ref, sd_ref):
    h = jnp.dot(x_ref[...], w_ref[...], preferred_element_type=jnp.float32)
    h_ref[...] = h
    ss_ref[...] = jnp.dot(h, as_ref[...], preferred_element_type=jnp.float32)
    sd_ref[...] = jnp.dot(h, ad_ref[...], preferred_element_type=jnp.float32)

  return pl.pallas_call(
      body,
      grid=(_N // _R,),
      in_specs=[
          pl.BlockSpec((_R, 128), lambda i: (i, 0)),
          pl.BlockSpec((128, 64), lambda i: (0, 0)),
          pl.BlockSpec((64, 16), lambda i: (0, 0)),
          pl.BlockSpec((64, 16), lambda i: (0, 0)),
      ],
      out_specs=[
          pl.BlockSpec((_R, 64), lambda i: (i, 0)),
          pl.BlockSpec((_R, 16), lambda i: (i, 0)),
          pl.BlockSpec((_R, 16), lambda i: (i, 0)),
      ],
      out_shape=[
          jax.ShapeDtypeStruct((_N, 64), jnp.float32),
          jax.ShapeDtypeStruct((_N, 16), jnp.float32),
          jax.ShapeDtypeStruct((_N, 16), jnp.float32),
      ],
  )(x, w1cat, asrc, adst)


def _elu(x):
  return jnp.where(x > 0, x, jnp.exp(jnp.minimum(x, 0.0)) - 1.0)


def _tc_c(accp, denp, brep, w2, a2t):
  def body(a_ref, d_ref, b_ref, w2_ref, a2_ref, th2_ref, s2s_ref, s2d_ref):
    acc = a_ref[0] + a_ref[1]
    den = jnp.sum(d_ref[...], axis=0)
    denw = jnp.dot(den, b_ref[...],
                   preferred_element_type=jnp.float32) + 1e-16
    h1 = _elu(acc / denw)
    h2h = jnp.dot(h1, w2_ref[...], preferred_element_type=jnp.float32)
    s2 = jnp.dot(h2h, a2_ref[...], preferred_element_type=jnp.float32)
    th2_ref[...] = h2h
    z = jnp.zeros((_RP, 15), jnp.float32)
    s2s_ref[...] = jnp.concatenate([s2[:, 0:1], z], axis=1)
    s2d_ref[...] = jnp.concatenate([s2[:, 1:2], z], axis=1)

  return pl.pallas_call(
      body,
      grid=(_NP // _RP,),
      in_specs=[
          pl.BlockSpec((_NC, _RP, 64), lambda i: (0, i, 0)),
          pl.BlockSpec((_NW, _RP, 8), lambda i: (0, i, 0)),
          pl.BlockSpec((8, 64), lambda i: (0, 0)),
          pl.BlockSpec((64, 16), lambda i: (0, 0)),
          pl.BlockSpec((16, 2), lambda i: (0, 0)),
      ],
      out_specs=[
          pl.BlockSpec((_RP, 16), lambda i: (i, 0)),
          pl.BlockSpec((_RP, 16), lambda i: (i, 0)),
          pl.BlockSpec((_RP, 16), lambda i: (i, 0)),
      ],
      out_shape=[
          jax.ShapeDtypeStruct((_NP, 16), jnp.float32),
          jax.ShapeDtypeStruct((_NP, 16), jnp.float32),
          jax.ShapeDtypeStruct((_NP, 16), jnp.float32),
      ],
  )(accp, denp, brep, w2, a2t)


def _tc_e(accp, denp):
  def body(a_ref, d_ref, o_ref):
    acc = a_ref[0] + a_ref[1]
    den = jnp.sum(d_ref[...], axis=0)
    h2 = _elu(acc / (den + 1e-16))
    m = jnp.max(h2, axis=1, keepdims=True)
    p = jnp.exp(h2 - m)
    o_ref[...] = p / jnp.sum(p, axis=1, keepdims=True)

  return pl.pallas_call(
      body,
      grid=(_NP // _RP,),
      in_specs=[
          pl.BlockSpec((_NC, _RP, 16), lambda i: (0, i, 0)),
          pl.BlockSpec((_NW, _RP, 1), lambda i: (0, i, 0)),
      ],
      out_specs=pl.BlockSpec((_RP, 16), lambda i: (i, 0)),
      out_shape=jax.ShapeDtypeStruct((_NP, 16), jnp.float32),
  )(accp, denp)


@jax.jit
def kernel(x, edge_index, W1, a1, W2, a2):
  src = edge_index[0].astype(jnp.int32)
  dst = edge_index[1].astype(jnp.int32)
  w1cat = jnp.transpose(W1, (1, 0, 2)).reshape(128, 64)
  eye = jnp.eye(8, dtype=jnp.float32)
  asrc = jnp.pad(
      jnp.einsum("kj,kl->kjl", a1[:, 0], eye).reshape(64, 8), ((0, 0), (0, 8)))
  adst = jnp.pad(
      jnp.einsum("kj,kl->kjl", a1[:, 1], eye).reshape(64, 8), ((0, 0), (0, 8)))
  brep = jnp.repeat(eye, 8, axis=1)

  h, ssrc, sdst = _tc_a(x, w1cat, asrc, adst)
  z64 = jnp.zeros((_NP, 64), jnp.float32)
  z16 = jnp.zeros((_NP, 16), jnp.float32)
  zd1 = jnp.zeros((_NP * 8,), jnp.float32)
  zd2 = jnp.zeros((_NP,), jnp.float32)
  accp, denp = _edge1(h, ssrc, sdst, src, dst, z64, zd1)
  denp = denp.reshape(_NW, _NP, 8)
  th2, s2s, s2d = _tc_c(accp, denp, brep, W2, a2.T)
  accp2, denp2 = _edge2(th2, s2s, s2d, src, dst, z16, zd2)
  denp2 = denp2.reshape(_NW, _NP, 1)
  return _tc_e(accp2, denp2)[:_N]


# speedup vs baseline: 1.9587x; 1.9587x over previous
import functools

import jax
import jax.numpy as jnp
from jax import lax
from jax.experimental import pallas as pl
from jax.experimental.pallas import tpu as pltpu
from jax.experimental.pallas import tpu_sc as plsc

_N = 10000
_E = 320000
_NC = 2
_NS = 16
_NW = _NC * _NS
_NP = 10240
_RPT = _NP // _NS


def _make_edge_pass(D, multi_head, _B, _unroll):
  _NBATCH = _E // _B
  mesh = plsc.VectorSubcoreMesh(core_axis_name="c", subcore_axis_name="s")

  @functools.partial(
      pl.kernel,
      out_type=(
          jax.ShapeDtypeStruct((_NC, _NP, D), jnp.float32),
          jax.ShapeDtypeStruct((_NC, _NP, 16), jnp.float32),
      ),
      mesh=mesh,
      compiler_params=pltpu.CompilerParams(use_tc_tiling_on_sc=False),
      scratch_types=(
          pltpu.VMEM_SHARED((_NP, D), jnp.float32),
          pltpu.VMEM_SHARED((_NP, 16), jnp.float32),
          pltpu.VMEM((_B,), jnp.int32),
          pltpu.VMEM((_B,), jnp.int32),
          pltpu.VMEM((_B, D), jnp.float32),
          pltpu.VMEM((_B, 16), jnp.float32),
          pltpu.VMEM((_B, 16), jnp.float32),
          pltpu.VMEM((_B, 16), jnp.float32),
          pltpu.VMEM((_B, D), jnp.float32),
          pltpu.SemaphoreType.DMA,
          pltpu.SemaphoreType.DMA,
          pltpu.SemaphoreType.DMA,
      ),
  )
  def kern(th, tssrc, tsdst, src, dst, zacc, zden, accp, denp,
           acc, den, idx_src, idx_dst, hrows, ssrc, sdst, w, orows,
           sem_h, sem_s, sem_d):
    cid = lax.axis_index("c")
    sid = lax.axis_index("s")
    wid = sid * _NC + cid
    iota = lax.iota(jnp.int32, 16)
    pat8 = iota >> 3

    r0 = sid * _RPT
    pltpu.sync_copy(zacc.at[pl.ds(r0, _RPT)], acc.at[pl.ds(r0, _RPT)])
    pltpu.sync_copy(zden.at[pl.ds(r0, _RPT)], den.at[pl.ds(r0, _RPT)])
    plsc.subcore_barrier()

    nb = (_NBATCH - wid + _NW - 1) // _NW

    def batch_body(i, carry):
      b = wid + i * _NW
      e0 = b * _B
      pltpu.sync_copy(src.at[pl.ds(e0, _B)], idx_src)
      pltpu.sync_copy(dst.at[pl.ds(e0, _B)], idx_dst)
      cp_h = pltpu.async_copy(th.at[idx_src], hrows, sem_h)
      cp_s = pltpu.async_copy(tssrc.at[idx_src], ssrc, sem_s)
      cp_d = pltpu.async_copy(tsdst.at[idx_dst], sdst, sem_d)
      cp_s.wait()
      cp_d.wait()
      cp_h.wait()

      def edge_body(eo, c2):
        for u in range(_unroll):
          e = eo * _unroll + u
          xv = ssrc[e] + sdst[e]
          wv = jnp.exp(jnp.maximum(xv, 0.2 * xv))
          w[e] = wv
          for c in range(D // 16):
            if multi_head:
              col = 2 * c + pat8
            else:
              col = jnp.zeros((16,), jnp.int32)
            wb = wv.at[col].get(mode="promise_in_bounds")
            orows[e, pl.ds(c * 16, 16)] = wb * hrows[e, pl.ds(c * 16, 16)]
        return c2

      lax.fori_loop(0, _B // _unroll, edge_body, 0)

      pltpu.sync_copy(orows, acc.at[idx_dst], add=True)
      pltpu.sync_copy(w, den.at[idx_dst], add=True)
      return carry

    lax.fori_loop(0, nb, batch_body, 0)
    plsc.subcore_barrier()

    pltpu.sync_copy(acc.at[pl.ds(r0, _RPT)], accp.at[cid, pl.ds(r0, _RPT)])
    pltpu.sync_copy(den.at[pl.ds(r0, _RPT)], denp.at[cid, pl.ds(r0, _RPT)])

  return kern


_edge1 = _make_edge_pass(64, True, 128, 4)
_edge2 = _make_edge_pass(16, False, 128, 8)

_R = 1000
_RP = 1024


def _tc_a(x, w1cat, asrc, adst):
  def body(x_ref, w_ref, as_ref, ad_ref, h_ref, ss_ref, sd_ref):
    h = jnp.dot(x_ref[...], w_ref[...], preferred_element_type=jnp.float32)
    h_ref[...] = h
    ss_ref[...] = jnp.dot(h, as_ref[...], preferred_element_type=jnp.float32)
    sd_ref[...] = jnp.dot(h, ad_ref[...], preferred_element_type=jnp.float32)

  return pl.pallas_call(
      body,
      grid=(_N // _R,),
      in_specs=[
          pl.BlockSpec((_R, 128), lambda i: (i, 0)),
          pl.BlockSpec((128, 64), lambda i: (0, 0)),
          pl.BlockSpec((64, 16), lambda i: (0, 0)),
          pl.BlockSpec((64, 16), lambda i: (0, 0)),
      ],
      out_specs=[
          pl.BlockSpec((_R, 64), lambda i: (i, 0)),
          pl.BlockSpec((_R, 16), lambda i: (i, 0)),
          pl.BlockSpec((_R, 16), lambda i: (i, 0)),
      ],
      out_shape=[
          jax.ShapeDtypeStruct((_N, 64), jnp.float32),
          jax.ShapeDtypeStruct((_N, 16), jnp.float32),
          jax.ShapeDtypeStruct((_N, 16), jnp.float32),
      ],
  )(x, w1cat, asrc, adst)


def _elu(x):
  return jnp.where(x > 0, x, jnp.exp(jnp.minimum(x, 0.0)) - 1.0)


def _tc_c(accp, denp, brep, w2, a2t):
  def body(a_ref, d_ref, b_ref, w2_ref, a2_ref, th2_ref, s2s_ref, s2d_ref):
    acc = a_ref[0] + a_ref[1]
    den = d_ref[0][:, 0:8] + d_ref[1][:, 0:8]
    denw = jnp.dot(den, b_ref[...],
                   preferred_element_type=jnp.float32) + 1e-16
    h1 = _elu(acc / denw)
    h2h = jnp.dot(h1, w2_ref[...], preferred_element_type=jnp.float32)
    s2 = jnp.dot(h2h, a2_ref[...], preferred_element_type=jnp.float32)
    th2_ref[...] = h2h
    z = jnp.zeros((_RP, 15), jnp.float32)
    s2s_ref[...] = jnp.concatenate([s2[:, 0:1], z], axis=1)
    s2d_ref[...] = jnp.concatenate([s2[:, 1:2], z], axis=1)

  return pl.pallas_call(
      body,
      grid=(_NP // _RP,),
      in_specs=[
          pl.BlockSpec((_NC, _RP, 64), lambda i: (0, i, 0)),
          pl.BlockSpec((_NC, _RP, 16), lambda i: (0, i, 0)),
          pl.BlockSpec((8, 64), lambda i: (0, 0)),
          pl.BlockSpec((64, 16), lambda i: (0, 0)),
          pl.BlockSpec((16, 2), lambda i: (0, 0)),
      ],
      out_specs=[
          pl.BlockSpec((_RP, 16), lambda i: (i, 0)),
          pl.BlockSpec((_RP, 16), lambda i: (i, 0)),
          pl.BlockSpec((_RP, 16), lambda i: (i, 0)),
      ],
      out_shape=[
          jax.ShapeDtypeStruct((_NP, 16), jnp.float32),
          jax.ShapeDtypeStruct((_NP, 16), jnp.float32),
          jax.ShapeDtypeStruct((_NP, 16), jnp.float32),
      ],
  )(accp, denp, brep, w2, a2t)


def _tc_e(accp, denp):
  def body(a_ref, d_ref, o_ref):
    acc = a_ref[0] + a_ref[1]
    den = d_ref[0][:, 0:1] + d_ref[1][:, 0:1]
    h2 = _elu(acc / (den + 1e-16))
    m = jnp.max(h2, axis=1, keepdims=True)
    p = jnp.exp(h2 - m)
    o_ref[...] = p / jnp.sum(p, axis=1, keepdims=True)

  return pl.pallas_call(
      body,
      grid=(_NP // _RP,),
      in_specs=[
          pl.BlockSpec((_NC, _RP, 16), lambda i: (0, i, 0)),
          pl.BlockSpec((_NC, _RP, 16), lambda i: (0, i, 0)),
      ],
      out_specs=pl.BlockSpec((_RP, 16), lambda i: (i, 0)),
      out_shape=jax.ShapeDtypeStruct((_NP, 16), jnp.float32),
  )(accp, denp)


@jax.jit
def kernel(x, edge_index, W1, a1, W2, a2):
  src = edge_index[0].astype(jnp.int32)
  dst = edge_index[1].astype(jnp.int32)
  w1cat = jnp.transpose(W1, (1, 0, 2)).reshape(128, 64)
  eye = jnp.eye(8, dtype=jnp.float32)
  asrc = jnp.pad(
      jnp.einsum("kj,kl->kjl", a1[:, 0], eye).reshape(64, 8), ((0, 0), (0, 8)))
  adst = jnp.pad(
      jnp.einsum("kj,kl->kjl", a1[:, 1], eye).reshape(64, 8), ((0, 0), (0, 8)))
  brep = jnp.repeat(eye, 8, axis=1)

  h, ssrc, sdst = _tc_a(x, w1cat, asrc, adst)
  z64 = jnp.zeros((_NP, 64), jnp.float32)
  z16 = jnp.zeros((_NP, 16), jnp.float32)
  accp, denp = _edge1(h, ssrc, sdst, src, dst, z64, z16)
  th2, s2s, s2d = _tc_c(accp, denp, brep, W2, a2.T)
  accp2, denp2 = _edge2(th2, s2s, s2d, src, dst, z16, z16)
  return _tc_e(accp2, denp2)[:_N]


# speedup vs baseline: 2.0599x; 1.0517x over previous
import functools

import jax
import jax.numpy as jnp
from jax import lax
from jax.experimental import pallas as pl
from jax.experimental.pallas import tpu as pltpu
from jax.experimental.pallas import tpu_sc as plsc

_N = 10000
_E = 320000
_NC = 2
_NS = 16
_NW = _NC * _NS
_NP = 10240
_RPT = _NP // _NS


_B = 128
_EPAD = 2560 * _B
_NBT = _EPAD // _B // _NW


def _make_edge_pass(D, multi_head, _unroll):
  mesh = plsc.VectorSubcoreMesh(core_axis_name="c", subcore_axis_name="s")

  @functools.partial(
      pl.kernel,
      out_type=(
          jax.ShapeDtypeStruct((_NC, _NP, D), jnp.float32),
          jax.ShapeDtypeStruct((_NC, _NP, 16), jnp.float32),
      ),
      mesh=mesh,
      compiler_params=pltpu.CompilerParams(use_tc_tiling_on_sc=False),
      scratch_types=(
          pltpu.VMEM_SHARED((_NP, D), jnp.float32),
          pltpu.VMEM_SHARED((_NP, 16), jnp.float32),
          [pltpu.VMEM((_B,), jnp.int32)] * 2,
          [pltpu.VMEM((_B,), jnp.int32)] * 2,
          [pltpu.VMEM((_B,), jnp.int32)] * 2,
          [pltpu.VMEM((_B, D), jnp.float32)] * 2,
          [pltpu.VMEM((_B, 16), jnp.float32)] * 2,
          [pltpu.VMEM((_B, 16), jnp.float32)] * 2,
          [pltpu.VMEM((_B, 16), jnp.float32)] * 2,
          [pltpu.VMEM((_B, D), jnp.float32)] * 2,
          [pltpu.SemaphoreType.DMA] * 2,
          [pltpu.SemaphoreType.DMA] * 2,
          [pltpu.SemaphoreType.DMA] * 2,
          [pltpu.SemaphoreType.DMA] * 2,
          [pltpu.SemaphoreType.DMA] * 2,
          [pltpu.SemaphoreType.DMA] * 2,
          [pltpu.SemaphoreType.DMA] * 2,
      ),
  )
  def kern(th, tssrc, tsdst, src, dst, zacc, zden, accp, denp,
           acc, den, isrc, idst, sidx, hr, ss, sd, w, orr,
           gh, gs, gd, qs, qd, sh, sw):
    cid = lax.axis_index("c")
    sid = lax.axis_index("s")
    wid = sid * _NC + cid
    iota = lax.iota(jnp.int32, 16)
    pat8 = iota >> 3

    r0 = sid * _RPT
    pltpu.sync_copy(zacc.at[pl.ds(r0, _RPT)], acc.at[pl.ds(r0, _RPT)])
    pltpu.sync_copy(zden.at[pl.ds(r0, _RPT)], den.at[pl.ds(r0, _RPT)])
    plsc.subcore_barrier()

    def issue(i, b):
      e0 = (wid + i * _NW) * _B
      pltpu.async_copy(src.at[pl.ds(e0, _B)], isrc[b], qs[b])
      pltpu.async_copy(dst.at[pl.ds(e0, _B)], idst[b], qd[b])
      pltpu.make_async_copy(src.at[pl.ds(0, _B)], isrc[b], qs[b]).wait()
      pltpu.make_async_copy(dst.at[pl.ds(0, _B)], idst[b], qd[b]).wait()
      pltpu.async_copy(th.at[isrc[b]], hr[b], gh[b])
      pltpu.async_copy(tssrc.at[isrc[b]], ss[b], gs[b])
      pltpu.async_copy(tsdst.at[idst[b]], sd[b], gd[b])

    def wait_gathers(b):
      pltpu.make_async_copy(th.at[pl.ds(0, _B)], hr[b], gh[b]).wait()
      pltpu.make_async_copy(tssrc.at[pl.ds(0, _B)], ss[b], gs[b]).wait()
      pltpu.make_async_copy(tsdst.at[pl.ds(0, _B)], sd[b], gd[b]).wait()

    def drain_scatters(b):
      del b

    def compute(b):
      def edge_body(eo, c2):
        for u in range(_unroll):
          e = eo * _unroll + u
          xv = ss[b][e] + sd[b][e]
          wv = jnp.exp(jnp.maximum(xv, 0.2 * xv))
          w[b][e] = wv
          for c in range(D // 16):
            if multi_head:
              col = 2 * c + pat8
            else:
              col = jnp.zeros((16,), jnp.int32)
            wb = wv.at[col].get(mode="promise_in_bounds")
            orr[b][e, pl.ds(c * 16, 16)] = wb * hr[b][e, pl.ds(c * 16, 16)]
        return c2

      lax.fori_loop(0, _B // _unroll, edge_body, 0)

    def start_scatters(b):
      pltpu.sync_copy(orr[b], acc.at[idst[b]], add=True)
      pltpu.sync_copy(w[b], den.at[idst[b]], add=True)

    for b in (0, 1):
      issue(b, b)

    @pl.loop(0, _NBT, step=2)
    def _pipeline(k0):
      for b in (0, 1):
        i = k0 + b

        @pl.when(i >= 2)
        def _():
          drain_scatters(b)

        wait_gathers(b)
        compute(b)
        start_scatters(b)

        @pl.when(i + 2 < _NBT)
        def _():
          issue(i + 2, b)

    for b in (0, 1):
      drain_scatters(b)
    plsc.subcore_barrier()

    pltpu.sync_copy(acc.at[pl.ds(r0, _RPT)], accp.at[cid, pl.ds(r0, _RPT)])
    pltpu.sync_copy(den.at[pl.ds(r0, _RPT)], denp.at[cid, pl.ds(r0, _RPT)])

  return kern


_edge1 = _make_edge_pass(64, True, 4)
_edge2 = _make_edge_pass(16, False, 8)

_R = 1000
_RP = 1024


def _tc_a(x, w1cat, asrc, adst):
  def body(x_ref, w_ref, as_ref, ad_ref, h_ref, ss_ref, sd_ref):
    h = jnp.dot(x_ref[...], w_ref[...], preferred_element_type=jnp.float32)
    h_ref[...] = h
    ss_ref[...] = jnp.dot(h, as_ref[...], preferred_element_type=jnp.float32)
    sd_ref[...] = jnp.dot(h, ad_ref[...], preferred_element_type=jnp.float32)

  return pl.pallas_call(
      body,
      grid=(_NP // _RP,),
      in_specs=[
          pl.BlockSpec((_RP, 128), lambda i: (i, 0)),
          pl.BlockSpec((128, 64), lambda i: (0, 0)),
          pl.BlockSpec((64, 16), lambda i: (0, 0)),
          pl.BlockSpec((64, 16), lambda i: (0, 0)),
      ],
      out_specs=[
          pl.BlockSpec((_RP, 64), lambda i: (i, 0)),
          pl.BlockSpec((_RP, 16), lambda i: (i, 0)),
          pl.BlockSpec((_RP, 16), lambda i: (i, 0)),
      ],
      out_shape=[
          jax.ShapeDtypeStruct((_NP, 64), jnp.float32),
          jax.ShapeDtypeStruct((_NP, 16), jnp.float32),
          jax.ShapeDtypeStruct((_NP, 16), jnp.float32),
      ],
  )(x, w1cat, asrc, adst)


def _elu(x):
  return jnp.where(x > 0, x, jnp.exp(jnp.minimum(x, 0.0)) - 1.0)


def _tc_c(accp, denp, brep, w2, a2t):
  def body(a_ref, d_ref, b_ref, w2_ref, a2_ref, th2_ref, s2s_ref, s2d_ref):
    acc = a_ref[0] + a_ref[1]
    den = d_ref[0][:, 0:8] + d_ref[1][:, 0:8]
    denw = jnp.dot(den, b_ref[...],
                   preferred_element_type=jnp.float32) + 1e-16
    h1 = _elu(acc / denw)
    h2h = jnp.dot(h1, w2_ref[...], preferred_element_type=jnp.float32)
    s2 = jnp.dot(h2h, a2_ref[...], preferred_element_type=jnp.float32)
    th2_ref[...] = h2h
    z = jnp.zeros((_RP, 15), jnp.float32)
    s2s_ref[...] = jnp.concatenate([s2[:, 0:1], z], axis=1)
    s2d_ref[...] = jnp.concatenate([s2[:, 1:2], z], axis=1)

  return pl.pallas_call(
      body,
      grid=(_NP // _RP,),
      in_specs=[
          pl.BlockSpec((_NC, _RP, 64), lambda i: (0, i, 0)),
          pl.BlockSpec((_NC, _RP, 16), lambda i: (0, i, 0)),
          pl.BlockSpec((8, 64), lambda i: (0, 0)),
          pl.BlockSpec((64, 16), lambda i: (0, 0)),
          pl.BlockSpec((16, 2), lambda i: (0, 0)),
      ],
      out_specs=[
          pl.BlockSpec((_RP, 16), lambda i: (i, 0)),
          pl.BlockSpec((_RP, 16), lambda i: (i, 0)),
          pl.BlockSpec((_RP, 16), lambda i: (i, 0)),
      ],
      out_shape=[
          jax.ShapeDtypeStruct((_NP, 16), jnp.float32),
          jax.ShapeDtypeStruct((_NP, 16), jnp.float32),
          jax.ShapeDtypeStruct((_NP, 16), jnp.float32),
      ],
  )(accp, denp, brep, w2, a2t)


def _tc_e(accp, denp):
  def body(a_ref, d_ref, o_ref):
    acc = a_ref[0] + a_ref[1]
    den = d_ref[0][:, 0:1] + d_ref[1][:, 0:1]
    h2 = _elu(acc / (den + 1e-16))
    m = jnp.max(h2, axis=1, keepdims=True)
    p = jnp.exp(h2 - m)
    o_ref[...] = p / jnp.sum(p, axis=1, keepdims=True)

  return pl.pallas_call(
      body,
      grid=(_NP // _RP,),
      in_specs=[
          pl.BlockSpec((_NC, _RP, 16), lambda i: (0, i, 0)),
          pl.BlockSpec((_NC, _RP, 16), lambda i: (0, i, 0)),
      ],
      out_specs=pl.BlockSpec((_RP, 16), lambda i: (i, 0)),
      out_shape=jax.ShapeDtypeStruct((_NP, 16), jnp.float32),
  )(accp, denp)


@jax.jit
def kernel(x, edge_index, W1, a1, W2, a2):
  pad = jnp.full((_EPAD - _E,), _NP - 1, jnp.int32)
  src = jnp.concatenate([edge_index[0].astype(jnp.int32), pad])
  dst = jnp.concatenate([edge_index[1].astype(jnp.int32), pad])
  x = jnp.concatenate([x, jnp.zeros((_NP - _N, 128), jnp.float32)], axis=0)
  w1cat = jnp.transpose(W1, (1, 0, 2)).reshape(128, 64)
  eye = jnp.eye(8, dtype=jnp.float32)
  asrc = jnp.pad(
      jnp.einsum("kj,kl->kjl", a1[:, 0], eye).reshape(64, 8), ((0, 0), (0, 8)))
  adst = jnp.pad(
      jnp.einsum("kj,kl->kjl", a1[:, 1], eye).reshape(64, 8), ((0, 0), (0, 8)))
  brep = jnp.repeat(eye, 8, axis=1)

  h, ssrc, sdst = _tc_a(x, w1cat, asrc, adst)
  z64 = jnp.zeros((_NP, 64), jnp.float32)
  z16 = jnp.zeros((_NP, 16), jnp.float32)
  accp, denp = _edge1(h, ssrc, sdst, src, dst, z64, z16)
  th2, s2s, s2d = _tc_c(accp, denp, brep, W2, a2.T)
  accp2, denp2 = _edge2(th2, s2s, s2d, src, dst, z16, z16)
  return _tc_e(accp2, denp2)[:_N]


# speedup vs baseline: 2.0684x; 1.0041x over previous
import functools

import jax
import jax.numpy as jnp
from jax import lax
from jax.experimental import pallas as pl
from jax.experimental.pallas import tpu as pltpu
from jax.experimental.pallas import tpu_sc as plsc

_N = 10000
_E = 320000
_NC = 2
_NS = 16
_NW = _NC * _NS
_NP = 10240
_RPT = _NP // _NS


_B = 128
_EPAD = 2560 * _B
_NBT = _EPAD // _B // _NW


def _make_edge_pass(D, SW, multi_head):
  mesh = plsc.VectorSubcoreMesh(core_axis_name="c", subcore_axis_name="s")
  sshape = (_B, SW) if multi_head else (_B,)
  tshape = (_NP, SW) if multi_head else (_NP,)
  dshape = (_NC, _NP, SW) if multi_head else (_NC, _NP)
  del SW

  @functools.partial(
      pl.kernel,
      out_type=(
          jax.ShapeDtypeStruct((_NC, _NP, D), jnp.float32),
          jax.ShapeDtypeStruct(dshape, jnp.float32),
      ),
      mesh=mesh,
      compiler_params=pltpu.CompilerParams(use_tc_tiling_on_sc=False),
      scratch_types=(
          pltpu.VMEM_SHARED((_NP, D), jnp.float32),
          pltpu.VMEM_SHARED(tshape, jnp.float32),
          [pltpu.VMEM((_B,), jnp.int32)] * 2,
          [pltpu.VMEM((_B,), jnp.int32)] * 2,
          [pltpu.VMEM((_B, D), jnp.float32)] * 2,
          [pltpu.VMEM(sshape, jnp.float32)] * 2,
          [pltpu.VMEM(sshape, jnp.float32)] * 2,
          [pltpu.VMEM(sshape, jnp.float32)] * 2,
          [pltpu.VMEM((_B, D), jnp.float32)] * 2,
          [pltpu.SemaphoreType.DMA] * 2,
          [pltpu.SemaphoreType.DMA] * 2,
          [pltpu.SemaphoreType.DMA] * 2,
          [pltpu.SemaphoreType.DMA] * 2,
          [pltpu.SemaphoreType.DMA] * 2,
      ),
  )
  def kern(th, tssrc, tsdst, src, dst, zacc, zden, accp, denp,
           acc, den, isrc, idst, hr, ss, sd, w, orr,
           gh, gs, gd, qs, qd):
    cid = lax.axis_index("c")
    sid = lax.axis_index("s")
    wid = sid * _NC + cid
    iota = lax.iota(jnp.int32, 16)
    pat8 = iota >> 3

    r0 = sid * _RPT
    pltpu.sync_copy(zacc.at[pl.ds(r0, _RPT)], acc.at[pl.ds(r0, _RPT)])
    pltpu.sync_copy(zden.at[pl.ds(r0, _RPT)], den.at[pl.ds(r0, _RPT)])
    plsc.subcore_barrier()

    def issue(i, b):
      e0 = (wid + i * _NW) * _B
      pltpu.async_copy(src.at[pl.ds(e0, _B)], isrc[b], qs[b])
      pltpu.async_copy(dst.at[pl.ds(e0, _B)], idst[b], qd[b])
      pltpu.make_async_copy(src.at[pl.ds(0, _B)], isrc[b], qs[b]).wait()
      pltpu.make_async_copy(dst.at[pl.ds(0, _B)], idst[b], qd[b]).wait()
      pltpu.async_copy(th.at[isrc[b]], hr[b], gh[b])
      pltpu.async_copy(tssrc.at[isrc[b]], ss[b], gs[b])
      pltpu.async_copy(tsdst.at[idst[b]], sd[b], gd[b])

    def wait_gathers(b):
      pltpu.make_async_copy(th.at[pl.ds(0, _B)], hr[b], gh[b]).wait()
      pltpu.make_async_copy(tssrc.at[pl.ds(0, _B)], ss[b], gs[b]).wait()
      pltpu.make_async_copy(tsdst.at[pl.ds(0, _B)], sd[b], gd[b]).wait()

    def compute_mh(b):
      cols = [2 * c + pat8 for c in range(D // 16)]

      def edge_body(eo, c2):
        for u in range(4):
          e = eo * 4 + u
          xv = ss[b][e] + sd[b][e]
          wv = jnp.exp(jnp.maximum(xv, 0.2 * xv))
          w[b][e] = wv
          for c in range(D // 16):
            wb = wv.at[cols[c]].get(mode="promise_in_bounds")
            orr[b][e, pl.ds(c * 16, 16)] = wb * hr[b][e, pl.ds(c * 16, 16)]
        return c2

      lax.fori_loop(0, _B // 4, edge_body, 0)

    def compute_sh(b):
      for v in range(_B // 16):
        o = v * 16
        xv = ss[b][pl.ds(o, 16)] + sd[b][pl.ds(o, 16)]
        w[b][pl.ds(o, 16)] = jnp.exp(jnp.maximum(xv, 0.2 * xv))
      splats = [jnp.full((16,), j, jnp.int32) for j in range(16)]

      def m_body(q, c2):
        wchunk = w[b][pl.ds(q * 16, 16)]
        for j in range(16):
          e = q * 16 + j
          wb = wchunk.at[splats[j]].get(mode="promise_in_bounds")
          orr[b][e, pl.ds(0, 16)] = wb * hr[b][e, pl.ds(0, 16)]
        return c2

      lax.fori_loop(0, _B // 16, m_body, 0)

    def start_scatters(b):
      pltpu.sync_copy(orr[b], acc.at[idst[b]], add=True)
      pltpu.sync_copy(w[b], den.at[idst[b]], add=True)

    for b in (0, 1):
      issue(b, b)

    @pl.loop(0, _NBT, step=2)
    def _pipeline(k0):
      for b in (0, 1):
        i = k0 + b
        wait_gathers(b)
        if multi_head:
          compute_mh(b)
        else:
          compute_sh(b)
        start_scatters(b)

        @pl.when(i + 2 < _NBT)
        def _():
          issue(i + 2, b)

    plsc.subcore_barrier()

    pltpu.sync_copy(acc.at[pl.ds(r0, _RPT)], accp.at[cid, pl.ds(r0, _RPT)])
    pltpu.sync_copy(den.at[pl.ds(r0, _RPT)], denp.at[cid, pl.ds(r0, _RPT)])

  return kern


_edge1 = _make_edge_pass(64, 16, True)
_edge2 = _make_edge_pass(16, 1, False)

_R = 1000
_RP = 1024


def _tc_a(x, w1cat, asrc, adst):
  def body(x_ref, w_ref, as_ref, ad_ref, h_ref, ss_ref, sd_ref):
    h = jnp.dot(x_ref[...], w_ref[...], preferred_element_type=jnp.float32)
    h_ref[...] = h
    ss_ref[...] = jnp.dot(h, as_ref[...], preferred_element_type=jnp.float32)
    sd_ref[...] = jnp.dot(h, ad_ref[...], preferred_element_type=jnp.float32)

  return pl.pallas_call(
      body,
      grid=(_NP // _RP,),
      in_specs=[
          pl.BlockSpec((_RP, 128), lambda i: (i, 0)),
          pl.BlockSpec((128, 64), lambda i: (0, 0)),
          pl.BlockSpec((64, 16), lambda i: (0, 0)),
          pl.BlockSpec((64, 16), lambda i: (0, 0)),
      ],
      out_specs=[
          pl.BlockSpec((_RP, 64), lambda i: (i, 0)),
          pl.BlockSpec((_RP, 16), lambda i: (i, 0)),
          pl.BlockSpec((_RP, 16), lambda i: (i, 0)),
      ],
      out_shape=[
          jax.ShapeDtypeStruct((_NP, 64), jnp.float32),
          jax.ShapeDtypeStruct((_NP, 16), jnp.float32),
          jax.ShapeDtypeStruct((_NP, 16), jnp.float32),
      ],
  )(x, w1cat, asrc, adst)


def _elu(x):
  return jnp.where(x > 0, x, jnp.exp(jnp.minimum(x, 0.0)) - 1.0)


def _tc_c(accp, denp, brep, w2, a2t):
  def body(a_ref, d_ref, b_ref, w2_ref, a2_ref, th2_ref, s2s_ref, s2d_ref):
    acc = a_ref[0] + a_ref[1]
    den = d_ref[0][:, 0:8] + d_ref[1][:, 0:8]
    denw = jnp.dot(den, b_ref[...],
                   preferred_element_type=jnp.float32) + 1e-16
    h1 = _elu(acc / denw)
    h2h = jnp.dot(h1, w2_ref[...], preferred_element_type=jnp.float32)
    s2 = jnp.dot(h2h, a2_ref[...], preferred_element_type=jnp.float32)
    th2_ref[...] = h2h
    s2s_ref[...] = s2[:, 0:1]
    s2d_ref[...] = s2[:, 1:2]

  return pl.pallas_call(
      body,
      grid=(_NP // _RP,),
      in_specs=[
          pl.BlockSpec((_NC, _RP, 64), lambda i: (0, i, 0)),
          pl.BlockSpec((_NC, _RP, 16), lambda i: (0, i, 0)),
          pl.BlockSpec((8, 64), lambda i: (0, 0)),
          pl.BlockSpec((64, 16), lambda i: (0, 0)),
          pl.BlockSpec((16, 2), lambda i: (0, 0)),
      ],
      out_specs=[
          pl.BlockSpec((_RP, 16), lambda i: (i, 0)),
          pl.BlockSpec((_RP, 1), lambda i: (i, 0)),
          pl.BlockSpec((_RP, 1), lambda i: (i, 0)),
      ],
      out_shape=[
          jax.ShapeDtypeStruct((_NP, 16), jnp.float32),
          jax.ShapeDtypeStruct((_NP, 1), jnp.float32),
          jax.ShapeDtypeStruct((_NP, 1), jnp.float32),
      ],
  )(accp, denp, brep, w2, a2t)


def _tc_e(accp, denp):
  def body(a_ref, d_ref, o_ref):
    acc = a_ref[0] + a_ref[1]
    den = d_ref[0] + d_ref[1]
    h2 = _elu(acc / (den + 1e-16))
    m = jnp.max(h2, axis=1, keepdims=True)
    p = jnp.exp(h2 - m)
    o_ref[...] = p / jnp.sum(p, axis=1, keepdims=True)

  return pl.pallas_call(
      body,
      grid=(_NP // _RP,),
      in_specs=[
          pl.BlockSpec((_NC, _RP, 16), lambda i: (0, i, 0)),
          pl.BlockSpec((_NC, _RP, 1), lambda i: (0, i, 0)),
      ],
      out_specs=pl.BlockSpec((_RP, 16), lambda i: (i, 0)),
      out_shape=jax.ShapeDtypeStruct((_NP, 16), jnp.float32),
  )(accp, denp)


@jax.jit
def kernel(x, edge_index, W1, a1, W2, a2):
  pad = jnp.full((_EPAD - _E,), _NP - 1, jnp.int32)
  src = jnp.concatenate([edge_index[0].astype(jnp.int32), pad])
  dst = jnp.concatenate([edge_index[1].astype(jnp.int32), pad])
  x = jnp.concatenate([x, jnp.zeros((_NP - _N, 128), jnp.float32)], axis=0)
  w1cat = jnp.transpose(W1, (1, 0, 2)).reshape(128, 64)
  eye = jnp.eye(8, dtype=jnp.float32)
  asrc = jnp.pad(
      jnp.einsum("kj,kl->kjl", a1[:, 0], eye).reshape(64, 8), ((0, 0), (0, 8)))
  adst = jnp.pad(
      jnp.einsum("kj,kl->kjl", a1[:, 1], eye).reshape(64, 8), ((0, 0), (0, 8)))
  brep = jnp.repeat(eye, 8, axis=1)

  h, ssrc, sdst = _tc_a(x, w1cat, asrc, adst)
  z64 = jnp.zeros((_NP, 64), jnp.float32)
  z16 = jnp.zeros((_NP, 16), jnp.float32)
  z1 = jnp.zeros((_NP,), jnp.float32)
  accp, denp = _edge1(h, ssrc, sdst, src, dst, z64, z16)
  th2, s2s, s2d = _tc_c(accp, denp, brep, W2, a2.T)
  accp2, denp2 = _edge2(th2, s2s.reshape(_NP), s2d.reshape(_NP), src, dst,
                        z16, z1)
  return _tc_e(accp2, denp2.reshape(_NC, _NP, 1))[:_N]


# speedup vs baseline: 2.0936x; 1.0122x over previous
import functools

import jax
import jax.numpy as jnp
from jax import lax
from jax.experimental import pallas as pl
from jax.experimental.pallas import tpu as pltpu
from jax.experimental.pallas import tpu_sc as plsc

_N = 10000
_E = 320000
_NC = 2
_NS = 16
_NW = _NC * _NS
_NP = 10240
_RPT = _NP // _NS


_B = 128
_EPAD = 2560 * _B
_NBT = _EPAD // _B // _NW


def _make_edge_pass(D, DW, multi_head):
  mesh = plsc.VectorSubcoreMesh(core_axis_name="c", subcore_axis_name="s")
  sshape = (_B, 16) if multi_head else (_B,)
  tshape = (_NP, 16) if multi_head else (_NP,)

  @functools.partial(
      pl.kernel,
      out_type=(jax.ShapeDtypeStruct((_NC, _NP, DW), jnp.float32),
                jax.ShapeDtypeStruct((_NC, _NP), jnp.float32)),
      mesh=mesh,
      compiler_params=pltpu.CompilerParams(use_tc_tiling_on_sc=False),
      scratch_types=(
          pltpu.VMEM_SHARED((_NP, DW), jnp.float32),
          pltpu.VMEM_SHARED((_NP,), jnp.float32),
          [pltpu.VMEM((_B,), jnp.int32)] * 2,
          [pltpu.VMEM((_B,), jnp.int32)] * 2,
          [pltpu.VMEM((_B, D), jnp.float32)] * 2,
          [pltpu.VMEM(sshape, jnp.float32)] * 2,
          [pltpu.VMEM(sshape, jnp.float32)] * 2,
          [pltpu.VMEM((_B,), jnp.float32)] * 2,
          [pltpu.VMEM((_B, DW), jnp.float32)] * 2,
          [pltpu.SemaphoreType.DMA] * 2,
          [pltpu.SemaphoreType.DMA] * 2,
          [pltpu.SemaphoreType.DMA] * 2,
          [pltpu.SemaphoreType.DMA] * 2,
          [pltpu.SemaphoreType.DMA] * 2,
      ),
  )
  def kern(th, tssrc, tsdst, src, dst, zacc, zden, accp, denp,
           acc, den, isrc, idst, hr, ss, sd, w, orr,
           gh, gs, gd, qs, qd):
    cid = lax.axis_index("c")
    sid = lax.axis_index("s")
    wid = sid * _NC + cid
    iota = lax.iota(jnp.int32, 16)
    pat8 = iota >> 3

    r0 = sid * _RPT
    pltpu.sync_copy(zacc.at[pl.ds(r0, _RPT)], acc.at[pl.ds(r0, _RPT)])
    pltpu.sync_copy(zden.at[pl.ds(r0, _RPT)], den.at[pl.ds(r0, _RPT)])
    plsc.subcore_barrier()

    def issue(i, b):
      e0 = (wid + i * _NW) * _B
      pltpu.async_copy(src.at[pl.ds(e0, _B)], isrc[b], qs[b])
      pltpu.async_copy(dst.at[pl.ds(e0, _B)], idst[b], qd[b])
      pltpu.make_async_copy(src.at[pl.ds(0, _B)], isrc[b], qs[b]).wait()
      pltpu.make_async_copy(dst.at[pl.ds(0, _B)], idst[b], qd[b]).wait()
      pltpu.async_copy(th.at[isrc[b]], hr[b], gh[b])
      pltpu.async_copy(tssrc.at[isrc[b]], ss[b], gs[b])
      pltpu.async_copy(tsdst.at[idst[b]], sd[b], gd[b])

    def wait_gathers(b):
      pltpu.make_async_copy(th.at[pl.ds(0, _B)], hr[b], gh[b]).wait()
      pltpu.make_async_copy(tssrc.at[pl.ds(0, _B)], ss[b], gs[b]).wait()
      pltpu.make_async_copy(tsdst.at[pl.ds(0, _B)], sd[b], gd[b]).wait()

    def compute_mh(b):
      cols = [2 * c + pat8 for c in range(D // 16)]

      def edge_body(eo, c2):
        for u in range(4):
          e = eo * 4 + u
          xv = ss[b][e] + sd[b][e]
          wv = jnp.exp(jnp.maximum(xv, 0.2 * xv))
          orr[b][e, pl.ds(0, 16)] = wv
          for c in range(D // 16):
            wb = wv.at[cols[c]].get(mode="promise_in_bounds")
            orr[b][e, pl.ds(8 + c * 16, 16)] = wb * hr[b][e, pl.ds(c * 16, 16)]
        return c2

      lax.fori_loop(0, _B // 4, edge_body, 0)

    def compute_sh(b):
      for v in range(_B // 16):
        o = v * 16
        xv = ss[b][pl.ds(o, 16)] + sd[b][pl.ds(o, 16)]
        w[b][pl.ds(o, 16)] = jnp.exp(jnp.maximum(xv, 0.2 * xv))
      splats = [jnp.full((16,), j, jnp.int32) for j in range(16)]

      def m_body(q, c2):
        wchunk = w[b][pl.ds(q * 16, 16)]
        for j in range(16):
          e = q * 16 + j
          wb = wchunk.at[splats[j]].get(mode="promise_in_bounds")
          orr[b][e, pl.ds(0, 16)] = wb * hr[b][e, pl.ds(0, 16)]
        return c2

      lax.fori_loop(0, _B // 16, m_body, 0)

    for b in (0, 1):
      issue(b, b)

    @pl.loop(0, _NBT, step=2)
    def _pipeline(k0):
      for b in (0, 1):
        i = k0 + b
        wait_gathers(b)
        if multi_head:
          compute_mh(b)
        else:
          compute_sh(b)
        pltpu.sync_copy(orr[b], acc.at[idst[b]], add=True)
        if not multi_head:
          pltpu.sync_copy(w[b], den.at[idst[b]], add=True)

        @pl.when(i + 2 < _NBT)
        def _():
          issue(i + 2, b)

    plsc.subcore_barrier()
    pltpu.sync_copy(acc.at[pl.ds(r0, _RPT)], accp.at[cid, pl.ds(r0, _RPT)])
    pltpu.sync_copy(den.at[pl.ds(r0, _RPT)], denp.at[cid, pl.ds(r0, _RPT)])

  return kern


_edge1 = _make_edge_pass(64, 72, True)
_edge2 = _make_edge_pass(16, 16, False)

_R = 1000
_RP = 1024


def _tc_a(x, w1cat, asrc, adst):
  def body(x_ref, w_ref, as_ref, ad_ref, h_ref, ss_ref, sd_ref):
    h = jnp.dot(x_ref[...], w_ref[...], preferred_element_type=jnp.float32)
    h_ref[...] = h
    ss_ref[...] = jnp.dot(h, as_ref[...], preferred_element_type=jnp.float32)
    sd_ref[...] = jnp.dot(h, ad_ref[...], preferred_element_type=jnp.float32)

  return pl.pallas_call(
      body,
      grid=(_NP // _RP,),
      in_specs=[
          pl.BlockSpec((_RP, 128), lambda i: (i, 0)),
          pl.BlockSpec((128, 64), lambda i: (0, 0)),
          pl.BlockSpec((64, 16), lambda i: (0, 0)),
          pl.BlockSpec((64, 16), lambda i: (0, 0)),
      ],
      out_specs=[
          pl.BlockSpec((_RP, 64), lambda i: (i, 0)),
          pl.BlockSpec((_RP, 16), lambda i: (i, 0)),
          pl.BlockSpec((_RP, 16), lambda i: (i, 0)),
      ],
      out_shape=[
          jax.ShapeDtypeStruct((_NP, 64), jnp.float32),
          jax.ShapeDtypeStruct((_NP, 16), jnp.float32),
          jax.ShapeDtypeStruct((_NP, 16), jnp.float32),
      ],
  )(x, w1cat, asrc, adst)


def _elu(x):
  return jnp.where(x > 0, x, jnp.exp(jnp.minimum(x, 0.0)) - 1.0)


def _tc_c(accp, pden, psel, w2, a2t):
  def body(a_ref, pd_ref, ps_ref, w2_ref, a2_ref, th2_ref, s2s_ref, s2d_ref):
    afull = a_ref[0] + a_ref[1]
    denw = jnp.dot(afull, pd_ref[...],
                   preferred_element_type=jnp.float32) + 1e-16
    acc = jnp.dot(afull, ps_ref[...], preferred_element_type=jnp.float32)
    h1 = _elu(acc / denw)
    h2h = jnp.dot(h1, w2_ref[...], preferred_element_type=jnp.float32)
    s2 = jnp.dot(h2h, a2_ref[...], preferred_element_type=jnp.float32)
    th2_ref[...] = h2h
    s2s_ref[...] = s2[:, 0:1]
    s2d_ref[...] = s2[:, 1:2]

  return pl.pallas_call(
      body,
      grid=(_NP // _RP,),
      in_specs=[
          pl.BlockSpec((_NC, _RP, 72), lambda i: (0, i, 0)),
          pl.BlockSpec((72, 64), lambda i: (0, 0)),
          pl.BlockSpec((72, 64), lambda i: (0, 0)),
          pl.BlockSpec((64, 16), lambda i: (0, 0)),
          pl.BlockSpec((16, 2), lambda i: (0, 0)),
      ],
      out_specs=[
          pl.BlockSpec((_RP, 16), lambda i: (i, 0)),
          pl.BlockSpec((_RP, 1), lambda i: (i, 0)),
          pl.BlockSpec((_RP, 1), lambda i: (i, 0)),
      ],
      out_shape=[
          jax.ShapeDtypeStruct((_NP, 16), jnp.float32),
          jax.ShapeDtypeStruct((_NP, 1), jnp.float32),
          jax.ShapeDtypeStruct((_NP, 1), jnp.float32),
      ],
  )(accp, pden, psel, w2, a2t)


def _tc_e(accp, denp):
  def body(a_ref, d_ref, o_ref):
    acc = a_ref[0] + a_ref[1]
    den = d_ref[0] + d_ref[1]
    h2 = _elu(acc / (den + 1e-16))
    m = jnp.max(h2, axis=1, keepdims=True)
    p = jnp.exp(h2 - m)
    o_ref[...] = p / jnp.sum(p, axis=1, keepdims=True)

  return pl.pallas_call(
      body,
      grid=(_NP // _RP,),
      in_specs=[
          pl.BlockSpec((_NC, _RP, 16), lambda i: (0, i, 0)),
          pl.BlockSpec((_NC, _RP, 1), lambda i: (0, i, 0)),
      ],
      out_specs=pl.BlockSpec((_RP, 16), lambda i: (i, 0)),
      out_shape=jax.ShapeDtypeStruct((_NP, 16), jnp.float32),
  )(accp, denp)


@jax.jit
def kernel(x, edge_index, W1, a1, W2, a2):
  pad = jnp.full((_EPAD - _E,), _NP - 1, jnp.int32)
  src = jnp.concatenate([edge_index[0].astype(jnp.int32), pad])
  dst = jnp.concatenate([edge_index[1].astype(jnp.int32), pad])
  x = jnp.concatenate([x, jnp.zeros((_NP - _N, 128), jnp.float32)], axis=0)
  w1cat = jnp.transpose(W1, (1, 0, 2)).reshape(128, 64)
  eye = jnp.eye(8, dtype=jnp.float32)
  asrc = jnp.pad(
      jnp.einsum("kj,kl->kjl", a1[:, 0], eye).reshape(64, 8), ((0, 0), (0, 8)))
  adst = jnp.pad(
      jnp.einsum("kj,kl->kjl", a1[:, 1], eye).reshape(64, 8), ((0, 0), (0, 8)))
  brep = jnp.repeat(eye, 8, axis=1)
  pden = jnp.pad(brep, ((0, 64), (0, 0)))
  psel = jnp.pad(jnp.eye(64, dtype=jnp.float32), ((8, 0), (0, 0)))


  h, ssrc, sdst = _tc_a(x, w1cat, asrc, adst)
  z72 = jnp.zeros((_NP, 72), jnp.float32)
  z16 = jnp.zeros((_NP, 16), jnp.float32)
  z1 = jnp.zeros((_NP,), jnp.float32)
  accp, _ = _edge1(h, ssrc, sdst, src, dst, z72, z1)
  th2, s2s, s2d = _tc_c(accp, pden, psel, W2, a2.T)
  accp2, denp2 = _edge2(th2, s2s.reshape(_NP), s2d.reshape(_NP), src, dst,
                        z16, z1)
  return _tc_e(accp2, denp2.reshape(_NC, _NP, 1))[:_N]
